# baseline (speedup 1.0000x reference)
import dataclasses

import jax
import jax.numpy as jnp
from jax.experimental import pallas as pl
from jax.experimental.pallas import tpu as pltpu
from jax.experimental.pallas import tpu_sc as plsc

_B, _NS, _NO = 4, 10475, 2048
_THRESHOLD = 0.1
_BIG = 1e30
_BS = 1024
_CN = 512
_NSP = 11264


def _compact_kernel(s4t_hbm, o4t_hbm, z_hbm, souts_hbm, oouts_hbm, cnt_hbm,
                    sinbuf, soutbuf, cntbuf, sem):
    c = jax.lax.axis_index("core")
    s = jax.lax.axis_index("subcore")
    tile = c * 16 + s

    @pl.when(tile < _B)
    def _():
        b = tile
        pltpu.async_copy(s4t_hbm.at[b], sinbuf.at[pl.ds(0, 4 * _NSP)],
                         sem).wait()
        pltpu.async_copy(z_hbm.at[pl.ds(0, 4 * _NSP)],
                         soutbuf.at[pl.ds(0, 4 * _NSP)], sem).wait()

        def chunk(i, cnt):
            w = sinbuf[pl.ds(3 * _NSP + i * 16, 16)]
            mask = w > _THRESHOLD
            mi = mask.astype(jnp.int32)
            pos = cnt + plsc.cumsum(mi) - 1
            pos = jnp.where(mask, pos, 0)
            base4 = pos * 4
            for coord in range(4):
                plsc.store_scatter(
                    soutbuf, [base4 + coord],
                    sinbuf[pl.ds(coord * _NSP + i * 16, 16)], mask=mask)
            return cnt + jnp.sum(mi)

        cnt = jax.lax.fori_loop(0, _NSP // 16, chunk, jnp.int32(0))
        cntbuf[...] = jnp.full((16,), cnt, jnp.int32)
        pltpu.async_copy(soutbuf.at[pl.ds(0, 4 * _NSP)], souts_hbm.at[b],
                         sem).wait()
        pltpu.async_copy(cntbuf, cnt_hbm.at[b, 0], sem).wait()

    @pl.when(jnp.logical_and(tile >= _B, tile < 2 * _B))
    def _():
        b = tile - _B
        pltpu.async_copy(o4t_hbm.at[b], sinbuf.at[pl.ds(0, 4 * _NO)],
                         sem).wait()
        pltpu.async_copy(z_hbm.at[pl.ds(0, 4 * _NO)],
                         soutbuf.at[pl.ds(0, 4 * _NO)], sem).wait()

        def chunk(i, cnt):
            w = sinbuf[pl.ds(3 * _NO + i * 16, 16)]
            mask = w > _THRESHOLD
            mi = mask.astype(jnp.int32)
            pos = cnt + plsc.cumsum(mi) - 1
            pos = jnp.where(mask, pos, 0)
            for coord in range(4):
                plsc.store_scatter(
                    soutbuf, [pos + coord * _NO],
                    sinbuf[pl.ds(coord * _NO + i * 16, 16)], mask=mask)
            return cnt + jnp.sum(mi)

        cnt = jax.lax.fori_loop(0, _NO // 16, chunk, jnp.int32(0))
        cntbuf[...] = jnp.full((16,), cnt, jnp.int32)
        pltpu.async_copy(soutbuf.at[pl.ds(0, 4 * _NO)], oouts_hbm.at[b],
                         sem).wait()
        pltpu.async_copy(cntbuf, cnt_hbm.at[b, 1], sem).wait()


def _sc_compact(s4t, o4t, zeros4):
    mesh = plsc.VectorSubcoreMesh(core_axis_name="core",
                                  subcore_axis_name="subcore")
    cp = pltpu.CompilerParams()
    if "needs_layout_passes" in pltpu.CompilerParams.__dataclass_fields__:
        cp = dataclasses.replace(cp, needs_layout_passes=False)
    return pl.kernel(
        _compact_kernel,
        out_type=[
            jax.ShapeDtypeStruct((_B, _NSP * 4), jnp.float32),
            jax.ShapeDtypeStruct((_B, _NO * 4), jnp.float32),
            jax.ShapeDtypeStruct((_B, 2, 16), jnp.int32),
        ],
        mesh=mesh,
        scratch_types=[
            pltpu.VMEM((4 * _NSP,), jnp.float32),
            pltpu.VMEM((4 * _NSP,), jnp.float32),
            pltpu.VMEM((16,), jnp.int32),
            pltpu.SemaphoreType.DMA,
        ],
        compiler_params=cp,
    )(s4t, o4t, zeros4)


def _cl_kernel(cnt_ref, s_ref, o_ref, loss_ref, valid_ref):
    b = pl.program_id(0)
    ns_c = cnt_ref[b, 0, 0]
    no_c = cnt_ref[b, 1, 0]

    ox = o_ref[0, 0:1, :]
    oy = o_ref[0, 1:2, :]
    oz = o_ref[0, 2:3, :]
    om_full = o_ref[0, 3:4, :]
    omask = om_full > _THRESHOLD
    o2 = ox * ox + oy * oy + oz * oz
    oxm2 = -2.0 * ox.astype(jnp.bfloat16).astype(jnp.float32)
    oym2 = -2.0 * oy.astype(jnp.bfloat16).astype(jnp.float32)
    ozm2 = -2.0 * oz.astype(jnp.bfloat16).astype(jnp.float32)
    om2m = jnp.where(omask, om_full * om_full, _BIG)

    def row_body(i, carry):
        hvec, oacc = carry
        rbase = i * _BS
        sc = s_ref[0, pl.ds(rbase, _BS), :]
        sx = sc[:, 0:1]
        sy = sc[:, 1:2]
        sz = sc[:, 2:3]
        sm = sc[:, 3:4]
        s2 = sx * sx + sy * sy + sz * sz
        rows = jax.lax.broadcasted_iota(jnp.int32, (_BS, 1), 0) + rbase
        rvalid = rows < ns_c
        sm2m = jnp.where(rvalid, sm * sm, _BIG)
        smw = jnp.where(rvalid, sm, 0.0)
        sxb = sx.astype(jnp.bfloat16).astype(jnp.float32)
        syb = sy.astype(jnp.bfloat16).astype(jnp.float32)
        szb = sz.astype(jnp.bfloat16).astype(jnp.float32)
        t = (((s2 + o2) + sxb * oxm2) + syb * oym2) + szb * ozm2
        d2 = jnp.maximum(t, 1e-12)
        v1 = d2 * om2m
        rmin = jnp.min(v1, axis=1, keepdims=True)
        hvec = hvec + smw * jnp.sqrt(rmin)
        v2 = d2 * sm2m
        oacc = jnp.minimum(oacc, jnp.min(v2, axis=0, keepdims=True))
        return hvec, oacc

    nrow = (ns_c + _BS - 1) // _BS
    init = (jnp.zeros((_BS, 1), jnp.float32),
            jnp.full((1, _NO), _BIG, jnp.float32))
    hvec, oacc = jax.lax.fori_loop(0, nrow, row_body, init)
    hsum = jnp.sum(hvec)
    ns = ns_c.astype(jnp.float32)
    no = no_c.astype(jnp.float32)

    osum = jnp.sum(jnp.where(omask, om_full * jnp.sqrt(oacc), 0.0))
    h2o_mean = hsum / jnp.maximum(ns, 1.0)
    o2h_mean = osum / jnp.maximum(no, 1.0)
    valid_b = jnp.logical_and(ns > 0, no > 0)
    contrib = jnp.where(valid_b, h2o_mean + o2h_mean, 0.0)

    loss_ref[...] = contrib.reshape(1, 1, 1)
    valid_ref[...] = valid_b.astype(jnp.float32).reshape(1, 1, 1)


def kernel(smplx_v, object_v, smpl_occlusion_maps, object_occlusion_maps,
           smpl_mean_occlusion_map, object_mean_occlusion_map):
    sm = smpl_occlusion_maps * smpl_mean_occlusion_map[None, :]
    om = object_occlusion_maps * object_mean_occlusion_map[None, :]

    s_all = jnp.concatenate([smplx_v, sm[:, :, None]], axis=2)
    s_all = jnp.pad(s_all, ((0, 0), (0, _NSP - _NS), (0, 0)))
    s4t = s_all.transpose(0, 2, 1).reshape(_B, 4 * _NSP)

    o4t = jnp.concatenate(
        [object_v.transpose(0, 2, 1), om[:, None, :]], axis=1)
    o4t = o4t.reshape(_B, 4 * _NO)

    zeros4 = jnp.zeros((_NSP * 4,), jnp.float32)

    s_comp, o_comp, counts = _sc_compact(s4t, o4t, zeros4)
    s_comp = s_comp.reshape(_B, _NSP, 4)
    o_comp = o_comp.reshape(_B, 4, _NO)

    loss, valid = pl.pallas_call(
        _cl_kernel,
        grid=(_B,),
        in_specs=[
            pl.BlockSpec(memory_space=pltpu.SMEM),
            pl.BlockSpec((1, _NSP, 4), lambda b: (b, 0, 0)),
            pl.BlockSpec((1, 4, _NO), lambda b: (b, 0, 0)),
        ],
        out_specs=[
            pl.BlockSpec((1, 1, 1), lambda b: (b, 0, 0)),
            pl.BlockSpec((1, 1, 1), lambda b: (b, 0, 0)),
        ],
        out_shape=[
            jax.ShapeDtypeStruct((_B, 1, 1), jnp.float32),
            jax.ShapeDtypeStruct((_B, 1, 1), jnp.float32),
        ],
    )(counts, s_comp, o_comp)

    total = jnp.sum(loss)
    count = jnp.sum(valid)
    return jnp.where(count > 0, total / jnp.maximum(count, 1.0), total)


# speedup vs baseline: 1.0725x; 1.0725x over previous
import dataclasses

import jax
import jax.numpy as jnp
from jax.experimental import pallas as pl
from jax.experimental.pallas import tpu as pltpu
from jax.experimental.pallas import tpu_sc as plsc

_B, _NS, _NO = 4, 10475, 2048
_THRESHOLD = 0.1
_BIG = 1e30
_BS = 704
_CN = 512
_NSP = 11264


def _compact_kernel(s4t_hbm, o4t_hbm, z_hbm, souts_hbm, oouts_hbm, cnt_hbm,
                    sinbuf, soutbuf, cntbuf, sem):
    c = jax.lax.axis_index("core")
    s = jax.lax.axis_index("subcore")
    tile = c * 16 + s

    @pl.when(tile < _B)
    def _():
        b = tile
        pltpu.async_copy(s4t_hbm.at[b], sinbuf.at[pl.ds(0, 4 * _NSP)],
                         sem).wait()
        pltpu.async_copy(z_hbm.at[pl.ds(0, 4 * _NSP)],
                         soutbuf.at[pl.ds(0, 4 * _NSP)], sem).wait()

        def chunk(i, cnt):
            w = sinbuf[pl.ds(3 * _NSP + i * 16, 16)]
            mask = w > _THRESHOLD
            mi = mask.astype(jnp.int32)
            pos = cnt + plsc.cumsum(mi) - 1
            pos = jnp.where(mask, pos, 0)
            base4 = pos * 4
            for coord in range(4):
                plsc.store_scatter(
                    soutbuf, [base4 + coord],
                    sinbuf[pl.ds(coord * _NSP + i * 16, 16)], mask=mask)
            return cnt + jnp.sum(mi)

        cnt = plsc.parallel_loop(0, _NSP // 16, carry=jnp.int32(0),
                                 unroll=4)(chunk)
        cntbuf[...] = jnp.full((16,), cnt, jnp.int32)
        pltpu.async_copy(soutbuf.at[pl.ds(0, 4 * _NSP)], souts_hbm.at[b],
                         sem).wait()
        pltpu.async_copy(cntbuf, cnt_hbm.at[b, 0], sem).wait()

    @pl.when(jnp.logical_and(tile >= _B, tile < 2 * _B))
    def _():
        b = tile - _B
        pltpu.async_copy(o4t_hbm.at[b], sinbuf.at[pl.ds(0, 4 * _NO)],
                         sem).wait()
        pltpu.async_copy(z_hbm.at[pl.ds(0, 4 * _NO)],
                         soutbuf.at[pl.ds(0, 4 * _NO)], sem).wait()

        def chunk(i, cnt):
            w = sinbuf[pl.ds(3 * _NO + i * 16, 16)]
            mask = w > _THRESHOLD
            mi = mask.astype(jnp.int32)
            pos = cnt + plsc.cumsum(mi) - 1
            pos = jnp.where(mask, pos, 0)
            for coord in range(4):
                plsc.store_scatter(
                    soutbuf, [pos + coord * _NO],
                    sinbuf[pl.ds(coord * _NO + i * 16, 16)], mask=mask)
            return cnt + jnp.sum(mi)

        cnt = plsc.parallel_loop(0, _NO // 16, carry=jnp.int32(0),
                                 unroll=4)(chunk)
        cntbuf[...] = jnp.full((16,), cnt, jnp.int32)
        pltpu.async_copy(soutbuf.at[pl.ds(0, 4 * _NO)], oouts_hbm.at[b],
                         sem).wait()
        pltpu.async_copy(cntbuf, cnt_hbm.at[b, 1], sem).wait()


def _sc_compact(s4t, o4t, zeros4):
    mesh = plsc.VectorSubcoreMesh(core_axis_name="core",
                                  subcore_axis_name="subcore")
    cp = pltpu.CompilerParams()
    if "needs_layout_passes" in pltpu.CompilerParams.__dataclass_fields__:
        cp = dataclasses.replace(cp, needs_layout_passes=False)
    return pl.kernel(
        _compact_kernel,
        out_type=[
            jax.ShapeDtypeStruct((_B, _NSP * 4), jnp.float32),
            jax.ShapeDtypeStruct((_B, _NO * 4), jnp.float32),
            jax.ShapeDtypeStruct((_B, 2, 16), jnp.int32),
        ],
        mesh=mesh,
        scratch_types=[
            pltpu.VMEM((4 * _NSP,), jnp.float32),
            pltpu.VMEM((4 * _NSP,), jnp.float32),
            pltpu.VMEM((16,), jnp.int32),
            pltpu.SemaphoreType.DMA,
        ],
        compiler_params=cp,
    )(s4t, o4t, zeros4)


def _cl_kernel(cnt_ref, s_ref, o_ref, loss_ref, valid_ref):
    b = pl.program_id(0)
    ns_c = cnt_ref[b, 0, 0]
    no_c = cnt_ref[b, 1, 0]

    ox = o_ref[0, 0:1, :]
    oy = o_ref[0, 1:2, :]
    oz = o_ref[0, 2:3, :]
    om_full = o_ref[0, 3:4, :]
    omask = om_full > _THRESHOLD
    o2 = ox * ox + oy * oy + oz * oz
    oxm2 = -2.0 * ox.astype(jnp.bfloat16).astype(jnp.float32)
    oym2 = -2.0 * oy.astype(jnp.bfloat16).astype(jnp.float32)
    ozm2 = -2.0 * oz.astype(jnp.bfloat16).astype(jnp.float32)
    om2m = jnp.where(omask, om_full * om_full, _BIG)

    def row_body(i, carry):
        hvec, oacc = carry
        rbase = i * _BS
        sc = s_ref[0, pl.ds(rbase, _BS), :]
        sx = sc[:, 0:1]
        sy = sc[:, 1:2]
        sz = sc[:, 2:3]
        sm = sc[:, 3:4]
        s2 = sx * sx + sy * sy + sz * sz
        rows = jax.lax.broadcasted_iota(jnp.int32, (_BS, 1), 0) + rbase
        rvalid = rows < ns_c
        sm2m = jnp.where(rvalid, sm * sm, _BIG)
        smw = jnp.where(rvalid, sm, 0.0)
        sxb = sx.astype(jnp.bfloat16).astype(jnp.float32)
        syb = sy.astype(jnp.bfloat16).astype(jnp.float32)
        szb = sz.astype(jnp.bfloat16).astype(jnp.float32)
        t = (((s2 + o2) + sxb * oxm2) + syb * oym2) + szb * ozm2
        d2 = jnp.maximum(t, 1e-12)
        v1 = d2 * om2m
        rmin = jnp.min(v1, axis=1, keepdims=True)
        hvec = hvec + smw * jnp.sqrt(rmin)
        v2 = d2 * sm2m
        oacc = jnp.minimum(oacc, jnp.min(v2, axis=0, keepdims=True))
        return hvec, oacc

    nrow = (ns_c + _BS - 1) // _BS
    init = (jnp.zeros((_BS, 1), jnp.float32),
            jnp.full((1, _NO), _BIG, jnp.float32))
    hvec, oacc = jax.lax.fori_loop(0, nrow, row_body, init)
    hsum = jnp.sum(hvec)
    ns = ns_c.astype(jnp.float32)
    no = no_c.astype(jnp.float32)

    osum = jnp.sum(jnp.where(omask, om_full * jnp.sqrt(oacc), 0.0))
    h2o_mean = hsum / jnp.maximum(ns, 1.0)
    o2h_mean = osum / jnp.maximum(no, 1.0)
    valid_b = jnp.logical_and(ns > 0, no > 0)
    contrib = jnp.where(valid_b, h2o_mean + o2h_mean, 0.0)

    loss_ref[...] = contrib.reshape(1, 1, 1)
    valid_ref[...] = valid_b.astype(jnp.float32).reshape(1, 1, 1)


def kernel(smplx_v, object_v, smpl_occlusion_maps, object_occlusion_maps,
           smpl_mean_occlusion_map, object_mean_occlusion_map):
    sm = smpl_occlusion_maps * smpl_mean_occlusion_map[None, :]
    om = object_occlusion_maps * object_mean_occlusion_map[None, :]

    s_all = jnp.concatenate([smplx_v, sm[:, :, None]], axis=2)
    s_all = jnp.pad(s_all, ((0, 0), (0, _NSP - _NS), (0, 0)))
    s4t = s_all.transpose(0, 2, 1).reshape(_B, 4 * _NSP)

    o4t = jnp.concatenate(
        [object_v.transpose(0, 2, 1), om[:, None, :]], axis=1)
    o4t = o4t.reshape(_B, 4 * _NO)

    zeros4 = jnp.zeros((_NSP * 4,), jnp.float32)

    s_comp, o_comp, counts = _sc_compact(s4t, o4t, zeros4)
    s_comp = s_comp.reshape(_B, _NSP, 4)
    o_comp = o_comp.reshape(_B, 4, _NO)

    loss, valid = pl.pallas_call(
        _cl_kernel,
        grid=(_B,),
        in_specs=[
            pl.BlockSpec(memory_space=pltpu.SMEM),
            pl.BlockSpec((1, _NSP, 4), lambda b: (b, 0, 0)),
            pl.BlockSpec((1, 4, _NO), lambda b: (b, 0, 0)),
        ],
        out_specs=[
            pl.BlockSpec((1, 1, 1), lambda b: (b, 0, 0)),
            pl.BlockSpec((1, 1, 1), lambda b: (b, 0, 0)),
        ],
        out_shape=[
            jax.ShapeDtypeStruct((_B, 1, 1), jnp.float32),
            jax.ShapeDtypeStruct((_B, 1, 1), jnp.float32),
        ],
    )(counts, s_comp, o_comp)

    total = jnp.sum(loss)
    count = jnp.sum(valid)
    return jnp.where(count > 0, total / jnp.maximum(count, 1.0), total)
